# baseline (speedup 1.0000x reference)
import functools

import jax
import jax.numpy as jnp
from jax import lax
from jax.experimental import pallas as pl
from jax.experimental.pallas import tpu as pltpu
from jax.experimental.pallas import tpu_sc as plsc

NW = 32
CHUNK = 128
BV = 2048
TB = 4096


def _tc_normalize_body(emb_t_ref, out_ref):
    x = emb_t_ref[...]
    d_dim = x.shape[0]
    s = jnp.sum(x * x, axis=0)
    z = x * lax.rsqrt(s)[None, :]
    z3 = z.reshape(d_dim, -1, 128 // d_dim)
    out_ref[...] = z3.transpose(1, 2, 0).reshape(-1, 128)


def _normalized_flat_table(emb_t, vocab, d_dim):
    grid = (vocab + BV - 1) // BV
    rows = BV * d_dim // 128
    return pl.pallas_call(
        _tc_normalize_body,
        grid=(grid,),
        in_specs=[pl.BlockSpec((d_dim, BV), lambda i: (0, i))],
        out_specs=pl.BlockSpec((rows, 128), lambda i: (i, 0)),
        out_shape=jax.ShapeDtypeStruct((grid * rows, 128), jnp.float32),
    )(emb_t)


def _tc_layout_body(flat_ref, out_ref):
    x = flat_ref[...]
    d_dim = out_ref.shape[1]
    t = x.reshape(-1, 128 // d_dim, d_dim).transpose(2, 0, 1)
    out_ref[...] = t.reshape(1, d_dim, -1)


def _to_output_layout(flat128, f_dim, b_dim, d_dim):
    nbk = b_dim // TB
    rows = TB * d_dim // 128
    return pl.pallas_call(
        _tc_layout_body,
        grid=(f_dim, nbk),
        in_specs=[pl.BlockSpec((rows, 128), lambda f, k: (f * nbk + k, 0))],
        out_specs=pl.BlockSpec((1, d_dim, TB), lambda f, k: (f, 0, k)),
        out_shape=jax.ShapeDtypeStruct((f_dim, d_dim, b_dim), jnp.float32),
    )(flat128)


def kernel(indices, embeddings):
    b_dim, f_dim = indices.shape
    vocab, d_dim = embeddings.shape
    flat_b = indices.size
    assert flat_b % (NW * CHUNK) == 0
    b_per_w = flat_b // NW
    n_chunks = b_per_w // CHUNK

    flat_table = _normalized_flat_table(embeddings.T, vocab, d_dim)
    pad_vocab = flat_table.shape[0] * 128 // d_dim
    table_n = flat_table.reshape(pad_vocab, d_dim)

    idx3 = indices.T.astype(jnp.int32).reshape(NW, n_chunks, CHUNK)

    mesh = plsc.VectorSubcoreMesh(core_axis_name="c", subcore_axis_name="s")

    @functools.partial(
        pl.kernel,
        mesh=mesh,
        compiler_params=pltpu.CompilerParams(
            needs_layout_passes=False, use_tc_tiling_on_sc=False),
        out_type=jax.ShapeDtypeStruct((flat_b, d_dim), jnp.float32),
        scratch_types=[
            pltpu.VMEM((n_chunks, CHUNK), jnp.int32),
            pltpu.VMEM((CHUNK, d_dim), jnp.float32),
            pltpu.SemaphoreType.DMA,
        ],
    )
    def run(table_hbm, idx_hbm, out_hbm, idx_v, buf, sem):
        wid = lax.axis_index("s") * 2 + lax.axis_index("c")
        pltpu.sync_copy(idx_hbm.at[wid], idx_v)

        def chunk_body(c, carry):
            pltpu.async_copy(table_hbm.at[idx_v.at[c]], buf, sem).wait()
            base = wid * b_per_w + c * CHUNK
            pltpu.sync_copy(buf, out_hbm.at[pl.ds(base, CHUNK)])
            return carry

        lax.fori_loop(0, n_chunks, chunk_body, 0)

    out = run(table_n, idx3)
    out128 = out.reshape(flat_b * d_dim // 128, 128)
    out_fdb = _to_output_layout(out128, f_dim, b_dim, d_dim)
    return out_fdb.transpose(2, 0, 1)


# speedup vs baseline: 3.8052x; 3.8052x over previous
import functools

import jax
import jax.numpy as jnp
from jax import lax
from jax.experimental import pallas as pl
from jax.experimental.pallas import tpu as pltpu
from jax.experimental.pallas import tpu_sc as plsc

NW = 32
CHUNK = 128
BV = 8192
TB = 16384


def _eye(n):
    r = lax.broadcasted_iota(jnp.int32, (n, n), 0)
    c = lax.broadcasted_iota(jnp.int32, (n, n), 1)
    return (r == c).astype(jnp.float32)


def _tc_normalize_body(emb_t_ref, out_ref):
    x = emb_t_ref[...]
    d_dim = x.shape[0]
    s = jnp.sum(x * x, axis=0)
    z = x * lax.rsqrt(s)[None, :]
    out_ref[...] = lax.dot_general(
        z, _eye(d_dim), (((0,), (0,)), ((), ())),
        preferred_element_type=jnp.float32)


def _normalized_table(emb_t, vocab, d_dim):
    grid = (vocab + BV - 1) // BV
    return pl.pallas_call(
        _tc_normalize_body,
        grid=(grid,),
        in_specs=[pl.BlockSpec((d_dim, BV), lambda i: (0, i))],
        out_specs=pl.BlockSpec((BV, d_dim), lambda i: (i, 0)),
        out_shape=jax.ShapeDtypeStruct((grid * BV, d_dim), jnp.float32),
    )(emb_t)


def _tc_layout_body(g_ref, out_ref):
    x = g_ref[...]
    d_dim = x.shape[1]
    t = lax.dot_general(
        _eye(d_dim), x, (((1,), (1,)), ((), ())),
        preferred_element_type=jnp.float32)
    out_ref[...] = t[None]


def _to_output_layout(flat, f_dim, b_dim, d_dim):
    nbk = b_dim // TB
    return pl.pallas_call(
        _tc_layout_body,
        grid=(f_dim, nbk),
        in_specs=[pl.BlockSpec((TB, d_dim), lambda f, k: (f * nbk + k, 0))],
        out_specs=pl.BlockSpec((1, d_dim, TB), lambda f, k: (f, 0, k)),
        out_shape=jax.ShapeDtypeStruct((f_dim, d_dim, b_dim), jnp.float32),
    )(flat)


def kernel(indices, embeddings):
    b_dim, f_dim = indices.shape
    vocab, d_dim = embeddings.shape
    flat_b = indices.size
    assert flat_b % (NW * CHUNK) == 0
    b_per_w = flat_b // NW
    n_chunks = b_per_w // CHUNK

    table_n = _normalized_table(embeddings.T, vocab, d_dim)

    idx3 = indices.T.astype(jnp.int32).reshape(NW, n_chunks, CHUNK)

    mesh = plsc.VectorSubcoreMesh(core_axis_name="c", subcore_axis_name="s")

    @functools.partial(
        pl.kernel,
        mesh=mesh,
        compiler_params=pltpu.CompilerParams(
            needs_layout_passes=False, use_tc_tiling_on_sc=False),
        out_type=jax.ShapeDtypeStruct((flat_b, d_dim), jnp.float32),
        scratch_types=[
            pltpu.VMEM((n_chunks, CHUNK), jnp.int32),
            pltpu.VMEM((CHUNK, d_dim), jnp.float32),
            pltpu.SemaphoreType.DMA,
        ],
    )
    def run(table_hbm, idx_hbm, out_hbm, idx_v, buf, sem):
        wid = lax.axis_index("s") * 2 + lax.axis_index("c")
        pltpu.sync_copy(idx_hbm.at[wid], idx_v)

        def chunk_body(c, carry):
            pltpu.async_copy(table_hbm.at[idx_v.at[c]], buf, sem).wait()
            base = wid * b_per_w + c * CHUNK
            pltpu.sync_copy(buf, out_hbm.at[pl.ds(base, CHUNK)])
            return carry

        lax.fori_loop(0, n_chunks, chunk_body, 0)

    out = run(table_n, idx3)
    out_fdb = _to_output_layout(out, f_dim, b_dim, d_dim)
    return out_fdb.transpose(2, 0, 1)


# speedup vs baseline: 3.9659x; 1.0422x over previous
import functools

import jax
import jax.numpy as jnp
from jax import lax
from jax.experimental import pallas as pl
from jax.experimental.pallas import tpu as pltpu
from jax.experimental.pallas import tpu_sc as plsc

NW = 32
CHUNK = 128
BV = 32768
TB = 16384


def _eye(n):
    r = lax.broadcasted_iota(jnp.int32, (n, n), 0)
    c = lax.broadcasted_iota(jnp.int32, (n, n), 1)
    return (r == c).astype(jnp.float32)


def _tc_normalize_body(emb_t_ref, out_ref):
    x = emb_t_ref[...]
    d_dim = x.shape[0]
    s = jnp.sum(x * x, axis=0)
    z = x * lax.rsqrt(s)[None, :]
    out_ref[...] = lax.dot_general(
        z, _eye(d_dim), (((0,), (0,)), ((), ())),
        preferred_element_type=jnp.float32)


def _normalized_table(emb_t, vocab, d_dim):
    grid = (vocab + BV - 1) // BV
    return pl.pallas_call(
        _tc_normalize_body,
        grid=(grid,),
        in_specs=[pl.BlockSpec((d_dim, BV), lambda i: (0, i))],
        out_specs=pl.BlockSpec((BV, d_dim), lambda i: (i, 0)),
        out_shape=jax.ShapeDtypeStruct((grid * BV, d_dim), jnp.float32),
    )(emb_t)


def _tc_layout_body(g_ref, out_ref):
    x = g_ref[...]
    d_dim = x.shape[1]
    t = lax.dot_general(
        _eye(d_dim), x, (((1,), (1,)), ((), ())),
        preferred_element_type=jnp.float32)
    out_ref[...] = t[None]


def _to_output_layout(flat, f_dim, b_dim, d_dim):
    nbk = b_dim // TB
    return pl.pallas_call(
        _tc_layout_body,
        grid=(f_dim, nbk),
        in_specs=[pl.BlockSpec((TB, d_dim), lambda f, k: (f * nbk + k, 0))],
        out_specs=pl.BlockSpec((1, d_dim, TB), lambda f, k: (f, 0, k)),
        out_shape=jax.ShapeDtypeStruct((f_dim, d_dim, b_dim), jnp.float32),
    )(flat)


def kernel(indices, embeddings):
    b_dim, f_dim = indices.shape
    vocab, d_dim = embeddings.shape
    flat_b = indices.size
    assert flat_b % (NW * CHUNK) == 0
    b_per_w = flat_b // NW
    n_chunks = b_per_w // CHUNK

    table_n = _normalized_table(embeddings.T, vocab, d_dim)

    idx3 = indices.T.astype(jnp.int32).reshape(NW, n_chunks, CHUNK)

    mesh = plsc.VectorSubcoreMesh(core_axis_name="c", subcore_axis_name="s")

    @functools.partial(
        pl.kernel,
        mesh=mesh,
        compiler_params=pltpu.CompilerParams(
            needs_layout_passes=False, use_tc_tiling_on_sc=False),
        out_type=jax.ShapeDtypeStruct((flat_b, d_dim), jnp.float32),
        scratch_types=[
            pltpu.VMEM((n_chunks, CHUNK), jnp.int32),
            pltpu.VMEM((CHUNK, d_dim), jnp.float32),
            pltpu.SemaphoreType.DMA,
        ],
    )
    def run(table_hbm, idx_hbm, out_hbm, idx_v, buf, sem):
        wid = lax.axis_index("s") * 2 + lax.axis_index("c")
        pltpu.sync_copy(idx_hbm.at[wid], idx_v)

        def chunk_body(c, carry):
            pltpu.async_copy(table_hbm.at[idx_v.at[c]], buf, sem).wait()
            base = wid * b_per_w + c * CHUNK
            pltpu.sync_copy(buf, out_hbm.at[pl.ds(base, CHUNK)])
            return carry

        lax.fori_loop(0, n_chunks, chunk_body, 0)

    out = run(table_n, idx3)
    out_fdb = _to_output_layout(out, f_dim, b_dim, d_dim)
    return out_fdb.transpose(2, 0, 1)


# speedup vs baseline: 4.0883x; 1.0309x over previous
import functools

import jax
import jax.numpy as jnp
from jax import lax
from jax.experimental import pallas as pl
from jax.experimental.pallas import tpu as pltpu
from jax.experimental.pallas import tpu_sc as plsc

NW = 32
CHUNK = 128
BV = 32768
TB = 16384


def _eye(n):
    r = lax.broadcasted_iota(jnp.int32, (n, n), 0)
    c = lax.broadcasted_iota(jnp.int32, (n, n), 1)
    return (r == c).astype(jnp.float32)


def _tc_normalize_body(emb_t_ref, out_ref):
    x = emb_t_ref[...]
    d_dim = x.shape[0]
    s = jnp.sum(x * x, axis=0)
    z = x * lax.rsqrt(s)[None, :]
    out_ref[...] = lax.dot_general(
        z, _eye(d_dim), (((0,), (0,)), ((), ())),
        preferred_element_type=jnp.float32)


def _normalized_table(emb_t, vocab, d_dim):
    grid = (vocab + BV - 1) // BV
    return pl.pallas_call(
        _tc_normalize_body,
        grid=(grid,),
        in_specs=[pl.BlockSpec((d_dim, BV), lambda i: (0, i))],
        out_specs=pl.BlockSpec((BV, d_dim), lambda i: (i, 0)),
        out_shape=jax.ShapeDtypeStruct((grid * BV, d_dim), jnp.float32),
    )(emb_t)


def _tc_layout_body(g_ref, out_ref):
    x = g_ref[...]
    d_dim = x.shape[1]
    t = lax.dot_general(
        _eye(d_dim), x, (((1,), (1,)), ((), ())),
        preferred_element_type=jnp.float32)
    out_ref[...] = t[None]


def _to_output_layout(flat, f_dim, b_dim, d_dim):
    nbk = b_dim // TB
    return pl.pallas_call(
        _tc_layout_body,
        grid=(f_dim, nbk),
        in_specs=[pl.BlockSpec((TB, d_dim), lambda f, k: (f * nbk + k, 0))],
        out_specs=pl.BlockSpec((1, d_dim, TB), lambda f, k: (f, 0, k)),
        out_shape=jax.ShapeDtypeStruct((f_dim, d_dim, b_dim), jnp.float32),
    )(flat)


def kernel(indices, embeddings):
    b_dim, f_dim = indices.shape
    vocab, d_dim = embeddings.shape
    flat_b = indices.size
    assert flat_b % (NW * CHUNK) == 0
    b_per_w = flat_b // NW
    n_chunks = b_per_w // CHUNK

    table_n = _normalized_table(embeddings.T, vocab, d_dim)

    idx3 = indices.T.astype(jnp.int32).reshape(NW, n_chunks, CHUNK)

    mesh = plsc.VectorSubcoreMesh(core_axis_name="c", subcore_axis_name="s")

    @functools.partial(
        pl.kernel,
        mesh=mesh,
        compiler_params=pltpu.CompilerParams(
            needs_layout_passes=False, use_tc_tiling_on_sc=False),
        out_type=jax.ShapeDtypeStruct((flat_b, d_dim), jnp.float32),
        scratch_types=[
            pltpu.VMEM((n_chunks, CHUNK), jnp.int32),
            pltpu.VMEM((CHUNK, d_dim), jnp.float32),
            pltpu.SemaphoreType.DMA,
        ],
    )
    def run(table_hbm, idx_hbm, out_hbm, idx_v, buf, sem):
        wid = lax.axis_index("s") * 2 + lax.axis_index("c")
        pltpu.sync_copy(idx_hbm.at[wid], idx_v)

        def chunk_body(c, carry):
            pltpu.async_copy(table_hbm.at[idx_v.at[c]], buf, sem).wait()
            base = wid * b_per_w + c * CHUNK
            pltpu.sync_copy(buf, out_hbm.at[pl.ds(base, CHUNK)])
            return carry

        lax.fori_loop(0, n_chunks, chunk_body, 0)

    out = run(table_n, idx3)
    return out.reshape(f_dim, b_dim, d_dim).transpose(1, 0, 2)


# speedup vs baseline: 4.4808x; 1.0960x over previous
import functools

import jax
import jax.numpy as jnp
from jax import lax
from jax.experimental import pallas as pl
from jax.experimental.pallas import tpu as pltpu
from jax.experimental.pallas import tpu_sc as plsc

NW = 32
CHUNK = 1024
BV = 32768
TB = 16384


def _eye(n):
    r = lax.broadcasted_iota(jnp.int32, (n, n), 0)
    c = lax.broadcasted_iota(jnp.int32, (n, n), 1)
    return (r == c).astype(jnp.float32)


def _tc_normalize_body(emb_t_ref, out_ref):
    x = emb_t_ref[...]
    d_dim = x.shape[0]
    s = jnp.sum(x * x, axis=0)
    z = x * lax.rsqrt(s)[None, :]
    out_ref[...] = lax.dot_general(
        z, _eye(d_dim), (((0,), (0,)), ((), ())),
        preferred_element_type=jnp.float32)


def _normalized_table(emb_t, vocab, d_dim):
    grid = (vocab + BV - 1) // BV
    return pl.pallas_call(
        _tc_normalize_body,
        grid=(grid,),
        in_specs=[pl.BlockSpec((d_dim, BV), lambda i: (0, i))],
        out_specs=pl.BlockSpec((BV, d_dim), lambda i: (i, 0)),
        out_shape=jax.ShapeDtypeStruct((grid * BV, d_dim), jnp.float32),
    )(emb_t)


def _tc_layout_body(g_ref, out_ref):
    x = g_ref[...]
    d_dim = x.shape[1]
    t = lax.dot_general(
        _eye(d_dim), x, (((1,), (1,)), ((), ())),
        preferred_element_type=jnp.float32)
    out_ref[...] = t[None]


def _to_output_layout(flat, f_dim, b_dim, d_dim):
    nbk = b_dim // TB
    return pl.pallas_call(
        _tc_layout_body,
        grid=(f_dim, nbk),
        in_specs=[pl.BlockSpec((TB, d_dim), lambda f, k: (f * nbk + k, 0))],
        out_specs=pl.BlockSpec((1, d_dim, TB), lambda f, k: (f, 0, k)),
        out_shape=jax.ShapeDtypeStruct((f_dim, d_dim, b_dim), jnp.float32),
    )(flat)


def kernel(indices, embeddings):
    b_dim, f_dim = indices.shape
    vocab, d_dim = embeddings.shape
    flat_b = indices.size
    assert flat_b % (NW * CHUNK) == 0
    b_per_w = flat_b // NW
    n_chunks = b_per_w // CHUNK

    table_n = _normalized_table(embeddings.T, vocab, d_dim)

    idx2 = indices.T.astype(jnp.int32).reshape(NW, b_per_w)

    mesh = plsc.VectorSubcoreMesh(core_axis_name="c", subcore_axis_name="s")

    @functools.partial(
        pl.kernel,
        mesh=mesh,
        compiler_params=pltpu.CompilerParams(
            needs_layout_passes=False, use_tc_tiling_on_sc=False),
        out_type=jax.ShapeDtypeStruct((flat_b, d_dim), jnp.float32),
        scratch_types=[
            pltpu.VMEM((b_per_w,), jnp.int32),
            pltpu.VMEM((CHUNK, d_dim), jnp.float32),
            pltpu.VMEM((CHUNK, d_dim), jnp.float32),
            pltpu.SemaphoreType.DMA,
            pltpu.SemaphoreType.DMA,
        ],
    )
    def run(table_hbm, idx_hbm, out_hbm, idx_v, buf0, buf1, sem0, sem1):
        wid = lax.axis_index("s") * 2 + lax.axis_index("c")
        pltpu.sync_copy(idx_hbm.at[wid], idx_v)
        bufs, sems = (buf0, buf1), (sem0, sem1)

        pend = [None, None]
        pend[0] = pltpu.async_copy(
            table_hbm.at[idx_v.at[pl.ds(0, CHUNK)]], bufs[0], sems[0])
        for c in range(n_chunks):
            cur = c % 2
            if c + 1 < n_chunks:
                nxt = (c + 1) % 2
                pend[nxt] = pltpu.async_copy(
                    table_hbm.at[idx_v.at[pl.ds((c + 1) * CHUNK, CHUNK)]],
                    bufs[nxt], sems[nxt])
            pend[cur].wait()
            base = wid * b_per_w + c * CHUNK
            pltpu.sync_copy(bufs[cur], out_hbm.at[pl.ds(base, CHUNK)])

    out = run(table_n, idx2)
    return out.reshape(f_dim, b_dim, d_dim).transpose(1, 0, 2)
